# Initial kernel scaffold; baseline (speedup 1.0000x reference)
import jax
import jax.numpy as jnp
from jax.experimental import pallas as pl


def kernel(X, Y, k, W):
    raise NotImplementedError("write your pallas kernel here")


# speedup vs baseline: 1.8683x; 1.8683x over previous
import jax
import jax.numpy as jnp
import numpy as np
from jax.experimental import pallas as pl
from jax.experimental.pallas import tpu as pltpu

_K = 32
_NEG = -9.0e15
_SENTINEL = -2.0


def _dot_nt(a, b, prec):
    return jax.lax.dot_general(
        a, b, (((1,), (1,)), ((), ())),
        precision=prec, preferred_element_type=jnp.float32)


def _topk_stats(v, rows):
    kf = jnp.float32(_K)

    def body(_, carry):
        v, slots, t, rs = carry
        m = jnp.max(v, axis=1, keepdims=True)
        eq = v == m
        cnt = jnp.sum(eq.astype(jnp.float32), axis=1, keepdims=True)
        active = slots < kf
        take = jnp.minimum(cnt, kf - slots)
        rs = rs + jnp.where(active, jnp.maximum(m, 0.0) * take, 0.0)
        t = jnp.where(active, m, t)
        slots = slots + jnp.where(active, cnt, 0.0)
        v = jnp.where(eq, _SENTINEL, v)
        return v, slots, t, rs

    init = (v,
            jnp.zeros((rows, 1), jnp.float32),
            jnp.full((rows, 1), _SENTINEL, jnp.float32),
            jnp.zeros((rows, 1), jnp.float32))
    _, _, t, rs = jax.lax.fori_loop(0, _K, body, init)
    return t, rs


def _topk_mask(v, rows):
    kf = jnp.float32(_K)

    def body(_, carry):
        v, slots, sel = carry
        m = jnp.max(v, axis=1, keepdims=True)
        eq = v == m
        cnt = jnp.sum(eq.astype(jnp.float32), axis=1, keepdims=True)
        active = slots < kf
        sel = jnp.maximum(sel, jnp.where(active & eq, 1.0, 0.0))
        slots = slots + jnp.where(active, cnt, 0.0)
        v = jnp.where(eq, _SENTINEL, v)
        return v, slots, sel

    init = (v,
            jnp.zeros((rows, 1), jnp.float32),
            jnp.zeros(v.shape, jnp.float32))
    _, _, sel = jax.lax.fori_loop(0, _K, body, init)
    return sel


def _phase_a_kernel(prec, R, x_ref, y_ref, w_ref,
                    xp_ref, yp_ref, n1_ref, n2_ref, t1_ref, rs_ref):
    i = pl.program_id(0)

    @pl.when(i == 0)
    def _():
        yp = jnp.dot(y_ref[...], w_ref[...], precision=prec,
                     preferred_element_type=jnp.float32)
        yp_ref[...] = yp
        n2_ref[...] = jnp.sqrt(jnp.sum(yp * yp, axis=1, keepdims=True))

    xp_r = jnp.dot(x_ref[...], w_ref[...], precision=prec,
                   preferred_element_type=jnp.float32)
    xp_ref[...] = xp_r
    n1_r = jnp.sqrt(jnp.sum(xp_r * xp_r, axis=1, keepdims=True))
    n1_ref[pl.ds(i * R, R), :] = n1_r
    mm = _dot_nt(xp_r, yp_ref[...], prec)
    n2t = jnp.reshape(n2_ref[...], (1, n2_ref.shape[0]))
    cos = mm / (n1_r * n2t + 1e-7)
    t, rs = _topk_stats(cos, R)
    t1_ref[pl.ds(i * R, R), :] = t
    rs_ref[pl.ds(i * R, R), :] = rs


def _phase_b_kernel(prec, R, c2, xp_ref, yp_ref, n1_ref, n2_ref,
                    t1_ref, dc_ref, dr_ref, s1_ref, s2_ref):
    i = pl.program_id(0)
    n = n1_ref.shape[0]

    xp_r = xp_ref[pl.ds(i * R, R), :]
    n1_r = n1_ref[pl.ds(i * R, R), :]
    n2t = jnp.reshape(n2_ref[...], (1, n))
    cos = _dot_nt(xp_r, yp_ref[...], prec) / (n1_r * n2t + 1e-7)
    t1_r = t1_ref[pl.ds(i * R, R), :]
    dc_r = dc_ref[pl.ds(i * R, R), :]
    keep = cos >= t1_r
    s1_ref[...] = jnp.where(keep, jnp.maximum(cos, 0.0), 0.0) * (dc_r * dr_ref[...])

    yp_r = yp_ref[pl.ds(i * R, R), :]
    n2_r = n2_ref[pl.ds(i * R, R), :]
    n1t = jnp.reshape(n1_ref[...], (1, n))
    cos_t = _dot_nt(yp_r, xp_ref[...], prec) / (n2_r * n1t + 1e-7)
    sel = _topk_mask(cos_t, R)
    s2_ref[...] = jnp.where(sel > 0.0, 0.0, c2)


def kernel(X, Y, k, W):
    del k
    n, d_in = X.shape
    R = 128
    grid = n // R
    prec = jax.lax.Precision.DEFAULT
    f32 = jnp.float32

    full = lambda shape: pl.BlockSpec(shape, lambda i: (0, 0))

    xp, yp, n1, n2, t1, rs = pl.pallas_call(
        lambda *refs: _phase_a_kernel(prec, R, *refs),
        grid=(grid,),
        in_specs=[pl.BlockSpec((R, d_in), lambda i: (i, 0)),
                  full(Y.shape), full(W.shape)],
        out_specs=[pl.BlockSpec((R, W.shape[1]), lambda i: (i, 0)),
                   full((n, W.shape[1])),
                   full((n, 1)), full((n, 1)), full((n, 1)), full((n, 1))],
        out_shape=[jax.ShapeDtypeStruct((n, W.shape[1]), f32),
                   jax.ShapeDtypeStruct((n, W.shape[1]), f32),
                   jax.ShapeDtypeStruct((n, 1), f32),
                   jax.ShapeDtypeStruct((n, 1), f32),
                   jax.ShapeDtypeStruct((n, 1), f32),
                   jax.ShapeDtypeStruct((n, 1), f32)],
    )(X, Y, W)

    dcol = rs ** -0.5
    dcol = jnp.where(jnp.isinf(dcol), 0.0, dcol)
    drow = jnp.reshape(dcol, (1, n))
    c2 = float(np.float32(_NEG) / (np.float32(_NEG) * np.float32(n - _K)))

    s1, s2 = pl.pallas_call(
        lambda *refs: _phase_b_kernel(prec, R, c2, *refs),
        grid=(grid,),
        in_specs=[full(xp.shape), full(yp.shape), full((n, 1)), full((n, 1)),
                  full((n, 1)), full((n, 1)), full((1, n))],
        out_specs=[pl.BlockSpec((R, n), lambda i: (i, 0)),
                   pl.BlockSpec((R, n), lambda i: (i, 0))],
        out_shape=[jax.ShapeDtypeStruct((n, n), f32),
                   jax.ShapeDtypeStruct((n, n), f32)],
    )(xp, yp, n1, n2, t1, dcol, drow)
    return (s1, s2)


# speedup vs baseline: 3.6267x; 1.9411x over previous
import jax
import jax.numpy as jnp
import numpy as np
from jax.experimental import pallas as pl
from jax.experimental.pallas import tpu as pltpu

_K = 32
_NEG = -9.0e15
_SENTINEL = -2.0


def _dot_nt(a, b, prec):
    return jax.lax.dot_general(
        a, b, (((1,), (1,)), ((), ())),
        precision=prec, preferred_element_type=jnp.float32)


def _topk_stats(v, rows):
    def body(_, carry):
        v, t, rs = carry
        m = jnp.max(v, axis=1, keepdims=True)
        rs = rs + jnp.maximum(m, 0.0)
        v = jnp.where(v == m, _SENTINEL, v)
        return v, m, rs

    init = (v,
            jnp.full((rows, 1), _SENTINEL, jnp.float32),
            jnp.zeros((rows, 1), jnp.float32))
    _, t, rs = jax.lax.fori_loop(0, _K, body, init)
    return t, rs


def _topk_extract(v):
    def body(_, v):
        m = jnp.max(v, axis=1, keepdims=True)
        return jnp.where(v == m, _SENTINEL, v)

    return jax.lax.fori_loop(0, _K, body, v)


def _phase_a_kernel(prec, R, x_ref, y_ref, w_ref,
                    xp_ref, yp_ref, n1_ref, n2_ref, t1_ref, rs_ref):
    i = pl.program_id(0)

    @pl.when(i == 0)
    def _():
        yp = jnp.dot(y_ref[...], w_ref[...], precision=prec,
                     preferred_element_type=jnp.float32)
        yp_ref[...] = yp
        n2_ref[...] = jnp.sqrt(jnp.sum(yp * yp, axis=1, keepdims=True))

    xp_r = jnp.dot(x_ref[...], w_ref[...], precision=prec,
                   preferred_element_type=jnp.float32)
    xp_ref[...] = xp_r
    n1_r = jnp.sqrt(jnp.sum(xp_r * xp_r, axis=1, keepdims=True))
    n1_ref[pl.ds(i * R, R), :] = n1_r
    mm = _dot_nt(xp_r, yp_ref[...], prec)
    n2t = jnp.reshape(n2_ref[...], (1, n2_ref.shape[0]))
    cos = mm / (n1_r * n2t + 1e-7)
    t, rs = _topk_stats(cos, R)
    t1_ref[pl.ds(i * R, R), :] = t
    rs_ref[pl.ds(i * R, R), :] = rs


def _phase_b_kernel(prec, R, c2, xp_ref, yp_ref, n1_ref, n2_ref,
                    t1_ref, dc_ref, dr_ref, s1_ref, s2_ref):
    i = pl.program_id(0)
    n = n1_ref.shape[0]

    xp_r = xp_ref[pl.ds(i * R, R), :]
    n1_r = n1_ref[pl.ds(i * R, R), :]
    n2t = jnp.reshape(n2_ref[...], (1, n))
    cos = _dot_nt(xp_r, yp_ref[...], prec) / (n1_r * n2t + 1e-7)
    t1_r = t1_ref[pl.ds(i * R, R), :]
    dc_r = dc_ref[pl.ds(i * R, R), :]
    keep = cos >= t1_r
    s1_ref[...] = jnp.where(keep, jnp.maximum(cos, 0.0), 0.0) * (dc_r * dr_ref[...])

    yp_r = yp_ref[pl.ds(i * R, R), :]
    n2_r = n2_ref[pl.ds(i * R, R), :]
    n1t = jnp.reshape(n1_ref[...], (1, n))
    cos_t = _dot_nt(yp_r, xp_ref[...], prec) / (n2_r * n1t + 1e-7)
    rem = _topk_extract(cos_t)
    s2_ref[...] = jnp.where(rem == _SENTINEL, 0.0, c2)


def kernel(X, Y, k, W):
    del k
    n, d_in = X.shape
    R = 128
    grid = n // R
    prec = jax.lax.Precision.DEFAULT
    f32 = jnp.float32

    full = lambda shape: pl.BlockSpec(shape, lambda i: (0, 0))

    xp, yp, n1, n2, t1, rs = pl.pallas_call(
        lambda *refs: _phase_a_kernel(prec, R, *refs),
        grid=(grid,),
        in_specs=[pl.BlockSpec((R, d_in), lambda i: (i, 0)),
                  full(Y.shape), full(W.shape)],
        out_specs=[pl.BlockSpec((R, W.shape[1]), lambda i: (i, 0)),
                   full((n, W.shape[1])),
                   full((n, 1)), full((n, 1)), full((n, 1)), full((n, 1))],
        out_shape=[jax.ShapeDtypeStruct((n, W.shape[1]), f32),
                   jax.ShapeDtypeStruct((n, W.shape[1]), f32),
                   jax.ShapeDtypeStruct((n, 1), f32),
                   jax.ShapeDtypeStruct((n, 1), f32),
                   jax.ShapeDtypeStruct((n, 1), f32),
                   jax.ShapeDtypeStruct((n, 1), f32)],
    )(X, Y, W)

    dcol = rs ** -0.5
    dcol = jnp.where(jnp.isinf(dcol), 0.0, dcol)
    drow = jnp.reshape(dcol, (1, n))
    c2 = float(np.float32(_NEG) / (np.float32(_NEG) * np.float32(n - _K)))

    s1, s2 = pl.pallas_call(
        lambda *refs: _phase_b_kernel(prec, R, c2, *refs),
        grid=(grid,),
        in_specs=[full(xp.shape), full(yp.shape), full((n, 1)), full((n, 1)),
                  full((n, 1)), full((n, 1)), full((1, n))],
        out_specs=[pl.BlockSpec((R, n), lambda i: (i, 0)),
                   pl.BlockSpec((R, n), lambda i: (i, 0))],
        out_shape=[jax.ShapeDtypeStruct((n, n), f32),
                   jax.ShapeDtypeStruct((n, n), f32)],
    )(xp, yp, n1, n2, t1, dcol, drow)
    return (s1, s2)


# speedup vs baseline: 7.0306x; 1.9386x over previous
import jax
import jax.numpy as jnp
import numpy as np
from jax.experimental import pallas as pl
from jax.experimental.pallas import tpu as pltpu

_K = 32
_NEG = -9.0e15
_SENTINEL = -2.0


def _dot_nt(a, b, prec):
    return jax.lax.dot_general(
        a, b, (((1,), (1,)), ((), ())),
        precision=prec, preferred_element_type=jnp.float32)


def _topk_stats(v, rows):
    def body(_, carry):
        v, t, rs = carry
        m = jnp.max(v, axis=1, keepdims=True)
        rs = rs + jnp.maximum(m, 0.0)
        v = jnp.where(v == m, _SENTINEL, v)
        return v, m, rs

    init = (v,
            jnp.full((rows, 1), _SENTINEL, jnp.float32),
            jnp.zeros((rows, 1), jnp.float32))
    _, t, rs = jax.lax.fori_loop(0, _K, body, init)
    return t, rs


_T_ROUNDS = 10
_CHUNKS = 32


def _chunk_candidates(v, rows, cols):
    v3 = jnp.reshape(v, (rows, _CHUNKS, cols // _CHUNKS))
    cands = []
    for t in range(_T_ROUNDS):
        cm = jnp.max(v3, axis=2, keepdims=True)
        cands.append(jnp.reshape(cm, (rows, _CHUNKS)))
        if t + 1 < _T_ROUNDS:
            v3 = jnp.where(v3 == cm, _SENTINEL, v3)
    return jnp.concatenate(cands, axis=1)


def _phase_a_kernel(prec, R, x_ref, y_ref, w_ref,
                    xp_ref, yp_ref, n1_ref, n2_ref, t1_ref, rs_ref):
    i = pl.program_id(0)

    @pl.when(i == 0)
    def _():
        yp = jnp.dot(y_ref[...], w_ref[...], precision=prec,
                     preferred_element_type=jnp.float32)
        yp_ref[...] = yp
        n2_ref[...] = jnp.sqrt(jnp.sum(yp * yp, axis=1, keepdims=True))

    xp_r = jnp.dot(x_ref[...], w_ref[...], precision=prec,
                   preferred_element_type=jnp.float32)
    xp_ref[...] = xp_r
    n1_r = jnp.sqrt(jnp.sum(xp_r * xp_r, axis=1, keepdims=True))
    n1_ref[pl.ds(i * R, R), :] = n1_r
    mm = _dot_nt(xp_r, yp_ref[...], prec)
    n2t = jnp.reshape(n2_ref[...], (1, n2_ref.shape[0]))
    cos = mm / (n1_r * n2t + 1e-7)
    cand = _chunk_candidates(cos, R, cos.shape[1])
    t, rs = _topk_stats(cand, R)
    t1_ref[pl.ds(i * R, R), :] = t
    rs_ref[pl.ds(i * R, R), :] = rs


def _phase_b_kernel(prec, R, c2, xp_ref, yp_ref, n1_ref, n2_ref,
                    t1_ref, dc_ref, dr_ref, s1_ref, s2_ref):
    i = pl.program_id(0)
    n = n1_ref.shape[0]

    xp_r = xp_ref[pl.ds(i * R, R), :]
    n1_r = n1_ref[pl.ds(i * R, R), :]
    n2t = jnp.reshape(n2_ref[...], (1, n))
    cos = _dot_nt(xp_r, yp_ref[...], prec) / (n1_r * n2t + 1e-7)
    t1_r = t1_ref[pl.ds(i * R, R), :]
    dc_r = dc_ref[pl.ds(i * R, R), :]
    keep = cos >= t1_r
    s1_ref[...] = jnp.where(keep, jnp.maximum(cos, 0.0), 0.0) * (dc_r * dr_ref[...])

    yp_r = yp_ref[pl.ds(i * R, R), :]
    n2_r = n2_ref[pl.ds(i * R, R), :]
    n1t = jnp.reshape(n1_ref[...], (1, n))
    cos_t = _dot_nt(yp_r, xp_ref[...], prec) / (n2_r * n1t + 1e-7)
    cand = _chunk_candidates(cos_t, R, cos_t.shape[1])
    t2, _ = _topk_stats(cand, R)
    s2_ref[...] = jnp.where(cos_t >= t2, 0.0, c2)


def kernel(X, Y, k, W):
    del k
    n, d_in = X.shape
    R = 128
    grid = n // R
    prec = jax.lax.Precision.DEFAULT
    f32 = jnp.float32

    full = lambda shape: pl.BlockSpec(shape, lambda i: (0, 0))

    xp, yp, n1, n2, t1, rs = pl.pallas_call(
        lambda *refs: _phase_a_kernel(prec, R, *refs),
        grid=(grid,),
        in_specs=[pl.BlockSpec((R, d_in), lambda i: (i, 0)),
                  full(Y.shape), full(W.shape)],
        out_specs=[pl.BlockSpec((R, W.shape[1]), lambda i: (i, 0)),
                   full((n, W.shape[1])),
                   full((n, 1)), full((n, 1)), full((n, 1)), full((n, 1))],
        out_shape=[jax.ShapeDtypeStruct((n, W.shape[1]), f32),
                   jax.ShapeDtypeStruct((n, W.shape[1]), f32),
                   jax.ShapeDtypeStruct((n, 1), f32),
                   jax.ShapeDtypeStruct((n, 1), f32),
                   jax.ShapeDtypeStruct((n, 1), f32),
                   jax.ShapeDtypeStruct((n, 1), f32)],
    )(X, Y, W)

    dcol = rs ** -0.5
    dcol = jnp.where(jnp.isinf(dcol), 0.0, dcol)
    drow = jnp.reshape(dcol, (1, n))
    c2 = float(np.float32(_NEG) / (np.float32(_NEG) * np.float32(n - _K)))

    s1, s2 = pl.pallas_call(
        lambda *refs: _phase_b_kernel(prec, R, c2, *refs),
        grid=(grid,),
        in_specs=[full(xp.shape), full(yp.shape), full((n, 1)), full((n, 1)),
                  full((n, 1)), full((n, 1)), full((1, n))],
        out_specs=[pl.BlockSpec((R, n), lambda i: (i, 0)),
                   pl.BlockSpec((R, n), lambda i: (i, 0))],
        out_shape=[jax.ShapeDtypeStruct((n, n), f32),
                   jax.ShapeDtypeStruct((n, n), f32)],
    )(xp, yp, n1, n2, t1, dcol, drow)
    return (s1, s2)


# speedup vs baseline: 8.6153x; 1.2254x over previous
import jax
import jax.numpy as jnp
import numpy as np
from jax.experimental import pallas as pl
from jax.experimental.pallas import tpu as pltpu

_K = 32
_NEG = -9.0e15
_SENTINEL = -2.0


def _dot_nt(a, b, prec):
    return jax.lax.dot_general(
        a, b, (((1,), (1,)), ((), ())),
        precision=prec, preferred_element_type=jnp.float32)


def _topk_stats(v, rows):
    def body(_, carry):
        v, t, rs = carry
        m = jnp.max(v, axis=1, keepdims=True)
        rs = rs + jnp.maximum(m, 0.0)
        v = jnp.where(v == m, _SENTINEL, v)
        return v, m, rs

    init = (v,
            jnp.full((rows, 1), _SENTINEL, jnp.float32),
            jnp.zeros((rows, 1), jnp.float32))
    _, t, rs = jax.lax.fori_loop(0, _K, body, init)
    return t, rs


_T_ROUNDS = 8
_CHUNKS = 32


def _chunk_candidates(v, rows, cols):
    v3 = jnp.reshape(v, (rows, _CHUNKS, cols // _CHUNKS))
    cands = []
    for t in range(_T_ROUNDS):
        cm = jnp.max(v3, axis=2, keepdims=True)
        cands.append(jnp.reshape(cm, (rows, _CHUNKS)))
        if t + 1 < _T_ROUNDS:
            v3 = jnp.where(v3 == cm, _SENTINEL, v3)
    return jnp.concatenate(cands, axis=1)


def _phase_a_kernel(prec, R, x_ref, y_ref, w_ref,
                    xp_ref, yp_ref, n1_ref, n2_ref, t1_ref, rs_ref):
    i = pl.program_id(0)

    @pl.when(i == 0)
    def _():
        yp = jnp.dot(y_ref[...], w_ref[...], precision=prec,
                     preferred_element_type=jnp.float32)
        yp_ref[...] = yp
        n2_ref[...] = jnp.sqrt(jnp.sum(yp * yp, axis=1, keepdims=True))

    xp_r = jnp.dot(x_ref[...], w_ref[...], precision=prec,
                   preferred_element_type=jnp.float32)
    xp_ref[...] = xp_r
    n1_r = jnp.sqrt(jnp.sum(xp_r * xp_r, axis=1, keepdims=True))
    n1_ref[pl.ds(i * R, R), :] = n1_r
    mm = _dot_nt(xp_r, yp_ref[...], prec)
    n2t = jnp.reshape(n2_ref[...], (1, n2_ref.shape[0]))
    cos = mm / (n1_r * n2t + 1e-7)
    cand = _chunk_candidates(cos, R, cos.shape[1])
    t, rs = _topk_stats(cand, R)
    t1_ref[pl.ds(i * R, R), :] = t
    rs_ref[pl.ds(i * R, R), :] = rs


def _phase_b_kernel(prec, R, c2, xp_ref, yp_ref, n1_ref, n2_ref,
                    t1_ref, dc_ref, dr_ref, s1_ref, s2_ref):
    i = pl.program_id(0)
    n = n1_ref.shape[0]

    xp_r = xp_ref[pl.ds(i * R, R), :]
    n1_r = n1_ref[pl.ds(i * R, R), :]
    n2t = jnp.reshape(n2_ref[...], (1, n))
    cos = _dot_nt(xp_r, yp_ref[...], prec) / (n1_r * n2t + 1e-7)
    t1_r = t1_ref[pl.ds(i * R, R), :]
    dc_r = dc_ref[pl.ds(i * R, R), :]
    keep = cos >= t1_r
    s1_ref[...] = jnp.where(keep, jnp.maximum(cos, 0.0), 0.0) * (dc_r * dr_ref[...])

    yp_r = yp_ref[pl.ds(i * R, R), :]
    n2_r = n2_ref[pl.ds(i * R, R), :]
    n1t = jnp.reshape(n1_ref[...], (1, n))
    cos_t = _dot_nt(yp_r, xp_ref[...], prec) / (n2_r * n1t + 1e-7)
    cand = _chunk_candidates(cos_t, R, cos_t.shape[1])
    t2, _ = _topk_stats(cand, R)
    s2_ref[...] = jnp.where(cos_t >= t2, 0.0, c2)


def kernel(X, Y, k, W):
    del k
    n, d_in = X.shape
    RA = 256
    R = 128
    prec = jax.lax.Precision.DEFAULT
    f32 = jnp.float32

    full = lambda shape: pl.BlockSpec(shape, lambda i: (0, 0))

    xp, yp, n1, n2, t1, rs = pl.pallas_call(
        lambda *refs: _phase_a_kernel(prec, RA, *refs),
        grid=(n // RA,),
        in_specs=[pl.BlockSpec((RA, d_in), lambda i: (i, 0)),
                  full(Y.shape), full(W.shape)],
        out_specs=[pl.BlockSpec((RA, W.shape[1]), lambda i: (i, 0)),
                   full((n, W.shape[1])),
                   full((n, 1)), full((n, 1)), full((n, 1)), full((n, 1))],
        out_shape=[jax.ShapeDtypeStruct((n, W.shape[1]), f32),
                   jax.ShapeDtypeStruct((n, W.shape[1]), f32),
                   jax.ShapeDtypeStruct((n, 1), f32),
                   jax.ShapeDtypeStruct((n, 1), f32),
                   jax.ShapeDtypeStruct((n, 1), f32),
                   jax.ShapeDtypeStruct((n, 1), f32)],
    )(X, Y, W)

    dcol = rs ** -0.5
    dcol = jnp.where(jnp.isinf(dcol), 0.0, dcol)
    drow = jnp.reshape(dcol, (1, n))
    c2 = float(np.float32(_NEG) / (np.float32(_NEG) * np.float32(n - _K)))

    s1, s2 = pl.pallas_call(
        lambda *refs: _phase_b_kernel(prec, R, c2, *refs),
        grid=(n // R,),
        in_specs=[full(xp.shape), full(yp.shape), full((n, 1)), full((n, 1)),
                  full((n, 1)), full((n, 1)), full((1, n))],
        out_specs=[pl.BlockSpec((R, n), lambda i: (i, 0)),
                   pl.BlockSpec((R, n), lambda i: (i, 0))],
        out_shape=[jax.ShapeDtypeStruct((n, n), f32),
                   jax.ShapeDtypeStruct((n, n), f32)],
    )(xp, yp, n1, n2, t1, dcol, drow)
    return (s1, s2)


# speedup vs baseline: 9.6576x; 1.1210x over previous
import jax
import jax.numpy as jnp
import numpy as np
from jax.experimental import pallas as pl
from jax.experimental.pallas import tpu as pltpu

_K = 32
_NEG = -9.0e15
_SENTINEL = -2.0


def _dot_nt(a, b, prec):
    return jax.lax.dot_general(
        a, b, (((1,), (1,)), ((), ())),
        precision=prec, preferred_element_type=jnp.float32)


def _topk_stats(v, rows):
    def body(_, carry):
        v, t, rs = carry
        m = jnp.max(v, axis=1, keepdims=True)
        rs = rs + jnp.maximum(m, 0.0)
        v = jnp.where(v == m, _SENTINEL, v)
        return v, m, rs

    init = (v,
            jnp.full((rows, 1), _SENTINEL, jnp.float32),
            jnp.zeros((rows, 1), jnp.float32))
    _, t, rs = jax.lax.fori_loop(0, _K, body, init)
    return t, rs


_T_ROUNDS = 8
_CHUNKS = 32


def _chunk_candidates(v, rows, cols):
    v3 = jnp.reshape(v, (rows, _CHUNKS, cols // _CHUNKS))
    cands = []
    for t in range(_T_ROUNDS):
        cm = jnp.max(v3, axis=2, keepdims=True)
        cands.append(jnp.reshape(cm, (rows, _CHUNKS)))
        if t + 1 < _T_ROUNDS:
            v3 = jnp.where(v3 == cm, _SENTINEL, v3)
    return jnp.concatenate(cands, axis=1)


def _phase_a_kernel(prec, R, x_ref, y_ref, w_ref,
                    xp_ref, yp_ref, n1_ref, n2_ref, t1_ref, rs_ref):
    i = pl.program_id(0)

    @pl.when(i == 0)
    def _():
        yp = jnp.dot(y_ref[...], w_ref[...], precision=prec,
                     preferred_element_type=jnp.float32)
        yp_ref[...] = yp
        n2_ref[...] = jnp.sqrt(jnp.sum(yp * yp, axis=1, keepdims=True))

    xp_r = jnp.dot(x_ref[...], w_ref[...], precision=prec,
                   preferred_element_type=jnp.float32)
    xp_ref[...] = xp_r
    n1_r = jnp.sqrt(jnp.sum(xp_r * xp_r, axis=1, keepdims=True))
    n1_ref[pl.ds(i * R, R), :] = n1_r
    mm = _dot_nt(xp_r, yp_ref[...], prec)
    n2t = jnp.reshape(n2_ref[...], (1, n2_ref.shape[0]))
    cos = mm / (n1_r * n2t + 1e-7)
    cand = _chunk_candidates(cos, R, cos.shape[1])
    t, rs = _topk_stats(cand, R)
    t1_ref[pl.ds(i * R, R), :] = t
    rs_ref[pl.ds(i * R, R), :] = rs


def _phase_b_kernel(prec, R, c2, xp_ref, yp_ref, stats_ref, dr_ref,
                    s1_ref, s2_ref):
    i = pl.program_id(0)
    n = stats_ref.shape[0]
    stats = stats_ref[...]
    n1 = stats[:, 0:1]
    n2 = stats[:, 1:2]
    stats_r = stats_ref[pl.ds(i * R, R), :]

    xp_r = xp_ref[pl.ds(i * R, R), :]
    n1_r = stats_r[:, 0:1]
    n2t = jnp.reshape(n2, (1, n))
    cos = _dot_nt(xp_r, yp_ref[...], prec) / (n1_r * n2t + 1e-7)
    t1_r = stats_r[:, 2:3]
    dc_r = stats_r[:, 3:4]
    keep = cos >= t1_r
    s1_ref[...] = jnp.where(keep, jnp.maximum(cos, 0.0), 0.0) * (dc_r * dr_ref[...])

    yp_r = yp_ref[pl.ds(i * R, R), :]
    n2_r = stats_r[:, 1:2]
    n1t = jnp.reshape(n1, (1, n))
    cos_t = _dot_nt(yp_r, xp_ref[...], prec) / (n2_r * n1t + 1e-7)
    cand = _chunk_candidates(cos_t, R, cos_t.shape[1])
    t2, _ = _topk_stats(cand, R)
    s2_ref[...] = jnp.where(cos_t >= t2, 0.0, c2)


def kernel(X, Y, k, W):
    del k
    n, d_in = X.shape
    RA = 256
    R = 256
    prec = jax.lax.Precision.DEFAULT
    f32 = jnp.float32

    full = lambda shape: pl.BlockSpec(shape, lambda i: (0, 0))

    xp, yp, n1, n2, t1, rs = pl.pallas_call(
        lambda *refs: _phase_a_kernel(prec, RA, *refs),
        grid=(n // RA,),
        in_specs=[pl.BlockSpec((RA, d_in), lambda i: (i, 0)),
                  full(Y.shape), full(W.shape)],
        out_specs=[pl.BlockSpec((RA, W.shape[1]), lambda i: (i, 0)),
                   full((n, W.shape[1])),
                   full((n, 1)), full((n, 1)), full((n, 1)), full((n, 1))],
        out_shape=[jax.ShapeDtypeStruct((n, W.shape[1]), f32),
                   jax.ShapeDtypeStruct((n, W.shape[1]), f32),
                   jax.ShapeDtypeStruct((n, 1), f32),
                   jax.ShapeDtypeStruct((n, 1), f32),
                   jax.ShapeDtypeStruct((n, 1), f32),
                   jax.ShapeDtypeStruct((n, 1), f32)],
    )(X, Y, W)

    dcol = rs ** -0.5
    dcol = jnp.where(jnp.isinf(dcol), 0.0, dcol)
    drow = jnp.reshape(dcol, (1, n))
    c2 = float(np.float32(_NEG) / (np.float32(_NEG) * np.float32(n - _K)))

    stats = jnp.concatenate([n1, n2, t1, dcol], axis=1)
    s1, s2 = pl.pallas_call(
        lambda *refs: _phase_b_kernel(prec, R, c2, *refs),
        grid=(n // R,),
        in_specs=[full(xp.shape), full(yp.shape), full((n, 4)), full((1, n))],
        out_specs=[pl.BlockSpec((R, n), lambda i: (i, 0)),
                   pl.BlockSpec((R, n), lambda i: (i, 0))],
        out_shape=[jax.ShapeDtypeStruct((n, n), f32),
                   jax.ShapeDtypeStruct((n, n), f32)],
    )(xp, yp, stats, drow)
    return (s1, s2)


# speedup vs baseline: 10.5339x; 1.0907x over previous
import jax
import jax.numpy as jnp
import numpy as np
from jax.experimental import pallas as pl
from jax.experimental.pallas import tpu as pltpu

_K = 32
_NEG = -9.0e15
_SENTINEL = -2.0


def _dot_nt(a, b, prec):
    return jax.lax.dot_general(
        a, b, (((1,), (1,)), ((), ())),
        precision=prec, preferred_element_type=jnp.float32)


def _topk_stats(v, rows):
    def body(_, carry):
        v, t, rs = carry
        m = jnp.max(v, axis=1, keepdims=True)
        rs = rs + jnp.maximum(m, 0.0)
        v = jnp.where(v == m, _SENTINEL, v)
        return v, m, rs

    init = (v,
            jnp.full((rows, 1), _SENTINEL, jnp.float32),
            jnp.zeros((rows, 1), jnp.float32))
    _, t, rs = jax.lax.fori_loop(0, _K, body, init)
    return t, rs


_T_ROUNDS = 6


def _chunk_candidates(v, rows, cols):
    v3 = jnp.reshape(v, (rows, cols // 128, 128))
    cands = []
    for t in range(_T_ROUNDS):
        cm = jnp.max(v3, axis=1, keepdims=True)
        cands.append(jnp.reshape(cm, (rows, 128)))
        if t + 1 < _T_ROUNDS:
            v3 = jnp.where(v3 == cm, _SENTINEL, v3)
    return jnp.concatenate(cands, axis=1)


def _phase_a_kernel(prec, R, x_ref, y_ref, w_ref,
                    xp_ref, yp_ref, n1_ref, n2_ref, t1_ref, rs_ref):
    i = pl.program_id(0)

    @pl.when(i == 0)
    def _():
        yp = jnp.dot(y_ref[...], w_ref[...], precision=prec,
                     preferred_element_type=jnp.float32)
        yp_ref[...] = yp
        n2_ref[...] = jnp.sqrt(jnp.sum(yp * yp, axis=1, keepdims=True))

    xp_r = jnp.dot(x_ref[...], w_ref[...], precision=prec,
                   preferred_element_type=jnp.float32)
    xp_ref[...] = xp_r
    n1_r = jnp.sqrt(jnp.sum(xp_r * xp_r, axis=1, keepdims=True))
    n1_ref[pl.ds(i * R, R), :] = n1_r
    mm = _dot_nt(xp_r, yp_ref[...], prec)
    n2t = jnp.reshape(n2_ref[...], (1, n2_ref.shape[0]))
    cos = mm / (n1_r * n2t + 1e-7)
    cand = _chunk_candidates(cos, R, cos.shape[1])
    t, rs = _topk_stats(cand, R)
    t1_ref[pl.ds(i * R, R), :] = t
    rs_ref[pl.ds(i * R, R), :] = rs


def _phase_b_kernel(prec, R, c2, xp_ref, yp_ref, stats_ref, dr_ref,
                    s1_ref, s2_ref):
    i = pl.program_id(0)
    n = stats_ref.shape[0]
    stats = stats_ref[...]
    n1 = stats[:, 0:1]
    n2 = stats[:, 1:2]
    stats_r = stats_ref[pl.ds(i * R, R), :]

    xp_r = xp_ref[pl.ds(i * R, R), :]
    n1_r = stats_r[:, 0:1]
    n2t = jnp.reshape(n2, (1, n))
    cos = _dot_nt(xp_r, yp_ref[...], prec) / (n1_r * n2t + 1e-7)
    t1_r = stats_r[:, 2:3]
    dc_r = stats_r[:, 3:4]
    keep = cos >= t1_r
    s1_ref[...] = jnp.where(keep, jnp.maximum(cos, 0.0), 0.0) * (dc_r * dr_ref[...])

    yp_r = yp_ref[pl.ds(i * R, R), :]
    n2_r = stats_r[:, 1:2]
    n1t = jnp.reshape(n1, (1, n))
    cos_t = _dot_nt(yp_r, xp_ref[...], prec) / (n2_r * n1t + 1e-7)
    cand = _chunk_candidates(cos_t, R, cos_t.shape[1])
    t2, _ = _topk_stats(cand, R)
    s2_ref[...] = jnp.where(cos_t >= t2, 0.0, c2)


def kernel(X, Y, k, W):
    del k
    n, d_in = X.shape
    RA = 256
    R = 256
    prec = jax.lax.Precision.DEFAULT
    f32 = jnp.float32

    full = lambda shape: pl.BlockSpec(shape, lambda i: (0, 0))

    xp, yp, n1, n2, t1, rs = pl.pallas_call(
        lambda *refs: _phase_a_kernel(prec, RA, *refs),
        grid=(n // RA,),
        in_specs=[pl.BlockSpec((RA, d_in), lambda i: (i, 0)),
                  full(Y.shape), full(W.shape)],
        out_specs=[pl.BlockSpec((RA, W.shape[1]), lambda i: (i, 0)),
                   full((n, W.shape[1])),
                   full((n, 1)), full((n, 1)), full((n, 1)), full((n, 1))],
        out_shape=[jax.ShapeDtypeStruct((n, W.shape[1]), f32),
                   jax.ShapeDtypeStruct((n, W.shape[1]), f32),
                   jax.ShapeDtypeStruct((n, 1), f32),
                   jax.ShapeDtypeStruct((n, 1), f32),
                   jax.ShapeDtypeStruct((n, 1), f32),
                   jax.ShapeDtypeStruct((n, 1), f32)],
    )(X, Y, W)

    dcol = rs ** -0.5
    dcol = jnp.where(jnp.isinf(dcol), 0.0, dcol)
    drow = jnp.reshape(dcol, (1, n))
    c2 = float(np.float32(_NEG) / (np.float32(_NEG) * np.float32(n - _K)))

    stats = jnp.concatenate([n1, n2, t1, dcol], axis=1)
    s1, s2 = pl.pallas_call(
        lambda *refs: _phase_b_kernel(prec, R, c2, *refs),
        grid=(n // R,),
        in_specs=[full(xp.shape), full(yp.shape), full((n, 4)), full((1, n))],
        out_specs=[pl.BlockSpec((R, n), lambda i: (i, 0)),
                   pl.BlockSpec((R, n), lambda i: (i, 0))],
        out_shape=[jax.ShapeDtypeStruct((n, n), f32),
                   jax.ShapeDtypeStruct((n, n), f32)],
    )(xp, yp, stats, drow)
    return (s1, s2)


# speedup vs baseline: 10.6812x; 1.0140x over previous
import jax
import jax.numpy as jnp
import numpy as np
from jax.experimental import pallas as pl
from jax.experimental.pallas import tpu as pltpu

_K = 32
_NEG = -9.0e15
_SENTINEL = -2.0


def _dot_nt(a, b, prec):
    return jax.lax.dot_general(
        a, b, (((1,), (1,)), ((), ())),
        precision=prec, preferred_element_type=jnp.float32)


def _topk_stats(v, rows):
    def body(_, carry):
        v, t, rs = carry
        m = jnp.max(v, axis=1, keepdims=True)
        rs = rs + jnp.maximum(m, 0.0)
        v = jnp.where(v == m, _SENTINEL, v)
        return v, m, rs

    init = (v,
            jnp.full((rows, 1), _SENTINEL, jnp.float32),
            jnp.zeros((rows, 1), jnp.float32))
    _, t, rs = jax.lax.fori_loop(0, _K, body, init)
    return t, rs


_T_ROUNDS = 7


def _chunk_candidates(v, rows, cols):
    v3 = jnp.reshape(v, (rows, cols // 128, 128))
    cands = []
    for t in range(_T_ROUNDS):
        cm = jnp.max(v3, axis=1, keepdims=True)
        cands.append(jnp.reshape(cm, (rows, 128)))
        if t + 1 < _T_ROUNDS:
            v3 = jnp.where(v3 == cm, _SENTINEL, v3)
    merged = [
        jnp.maximum(cands[t][:, 0:64], cands[_T_ROUNDS - 1 - t][:, 64:128])
        for t in range(_T_ROUNDS)
    ]
    return jnp.concatenate(merged, axis=1)


def _phase_a_kernel(prec, R, x_ref, y_ref, w_ref,
                    xp_ref, yp_ref, n1_ref, n2_ref, t1_ref, rs_ref):
    i = pl.program_id(0)

    @pl.when(i == 0)
    def _():
        yp = jnp.dot(y_ref[...], w_ref[...], precision=prec,
                     preferred_element_type=jnp.float32)
        yp_ref[...] = yp
        n2_ref[...] = jnp.sqrt(jnp.sum(yp * yp, axis=1, keepdims=True))

    xp_r = jnp.dot(x_ref[...], w_ref[...], precision=prec,
                   preferred_element_type=jnp.float32)
    xp_ref[...] = xp_r
    n1_r = jnp.sqrt(jnp.sum(xp_r * xp_r, axis=1, keepdims=True))
    n1_ref[pl.ds(i * R, R), :] = n1_r
    mm = _dot_nt(xp_r, yp_ref[...], prec)
    n2t = jnp.reshape(n2_ref[...], (1, n2_ref.shape[0]))
    cos = mm / (n1_r * n2t + 1e-7)
    cand = _chunk_candidates(cos, R, cos.shape[1])
    t, rs = _topk_stats(cand, R)
    t1_ref[pl.ds(i * R, R), :] = t
    rs_ref[pl.ds(i * R, R), :] = rs


def _phase_b_kernel(prec, R, c2, xp_ref, yp_ref, stats_ref, dr_ref,
                    s1_ref, s2_ref):
    i = pl.program_id(0)
    n = stats_ref.shape[0]
    stats = stats_ref[...]
    n1 = stats[:, 0:1]
    n2 = stats[:, 1:2]
    stats_r = stats_ref[pl.ds(i * R, R), :]

    xp_r = xp_ref[pl.ds(i * R, R), :]
    n1_r = stats_r[:, 0:1]
    n2t = jnp.reshape(n2, (1, n))
    cos = _dot_nt(xp_r, yp_ref[...], prec) / (n1_r * n2t + 1e-7)
    t1_r = stats_r[:, 2:3]
    dc_r = stats_r[:, 3:4]
    keep = cos >= t1_r
    s1_ref[...] = jnp.where(keep, jnp.maximum(cos, 0.0), 0.0) * (dc_r * dr_ref[...])

    yp_r = yp_ref[pl.ds(i * R, R), :]
    n2_r = stats_r[:, 1:2]
    n1t = jnp.reshape(n1, (1, n))
    cos_t = _dot_nt(yp_r, xp_ref[...], prec) / (n2_r * n1t + 1e-7)
    cand = _chunk_candidates(cos_t, R, cos_t.shape[1])
    t2, _ = _topk_stats(cand, R)
    s2_ref[...] = jnp.where(cos_t >= t2, 0.0, c2)


def kernel(X, Y, k, W):
    del k
    n, d_in = X.shape
    RA = 256
    R = 256
    prec = jax.lax.Precision.DEFAULT
    f32 = jnp.float32

    full = lambda shape: pl.BlockSpec(shape, lambda i: (0, 0))

    xp, yp, n1, n2, t1, rs = pl.pallas_call(
        lambda *refs: _phase_a_kernel(prec, RA, *refs),
        grid=(n // RA,),
        in_specs=[pl.BlockSpec((RA, d_in), lambda i: (i, 0)),
                  full(Y.shape), full(W.shape)],
        out_specs=[pl.BlockSpec((RA, W.shape[1]), lambda i: (i, 0)),
                   full((n, W.shape[1])),
                   full((n, 1)), full((n, 1)), full((n, 1)), full((n, 1))],
        out_shape=[jax.ShapeDtypeStruct((n, W.shape[1]), f32),
                   jax.ShapeDtypeStruct((n, W.shape[1]), f32),
                   jax.ShapeDtypeStruct((n, 1), f32),
                   jax.ShapeDtypeStruct((n, 1), f32),
                   jax.ShapeDtypeStruct((n, 1), f32),
                   jax.ShapeDtypeStruct((n, 1), f32)],
    )(X, Y, W)

    dcol = rs ** -0.5
    dcol = jnp.where(jnp.isinf(dcol), 0.0, dcol)
    drow = jnp.reshape(dcol, (1, n))
    c2 = float(np.float32(_NEG) / (np.float32(_NEG) * np.float32(n - _K)))

    stats = jnp.concatenate([n1, n2, t1, dcol], axis=1)
    s1, s2 = pl.pallas_call(
        lambda *refs: _phase_b_kernel(prec, R, c2, *refs),
        grid=(n // R,),
        in_specs=[full(xp.shape), full(yp.shape), full((n, 4)), full((1, n))],
        out_specs=[pl.BlockSpec((R, n), lambda i: (i, 0)),
                   pl.BlockSpec((R, n), lambda i: (i, 0))],
        out_shape=[jax.ShapeDtypeStruct((n, n), f32),
                   jax.ShapeDtypeStruct((n, n), f32)],
    )(xp, yp, stats, drow)
    return (s1, s2)


# speedup vs baseline: 11.5093x; 1.0775x over previous
import jax
import jax.numpy as jnp
import numpy as np
from jax.experimental import pallas as pl
from jax.experimental.pallas import tpu as pltpu

_K = 32
_NEG = -9.0e15
_SENTINEL = -2.0


def _dot_nt(a, b, prec):
    return jax.lax.dot_general(
        a, b, (((1,), (1,)), ((), ())),
        precision=prec, preferred_element_type=jnp.float32)


def _topk_stats(v, rows):
    def body(_, carry):
        v, t, rs = carry
        m = jnp.max(v, axis=1, keepdims=True)
        rs = rs + jnp.maximum(m, 0.0)
        v = jnp.where(v == m, _SENTINEL, v)
        return v, m, rs

    init = (v,
            jnp.full((rows, 1), _SENTINEL, jnp.float32),
            jnp.zeros((rows, 1), jnp.float32))
    _, t, rs = jax.lax.fori_loop(0, _K, body, init)
    return t, rs


_T_ROUNDS = 7


def _chunk_candidates(v, rows, cols):
    v3 = jnp.reshape(v, (rows, cols // 128, 128))
    cands = []
    for t in range(_T_ROUNDS):
        cm = jnp.max(v3, axis=1, keepdims=True)
        cands.append(jnp.reshape(cm, (rows, 128)))
        if t + 1 < _T_ROUNDS:
            v3 = jnp.where(v3 == cm, _SENTINEL, v3)
    merged = [
        jnp.maximum(cands[t][:, 0:64], cands[_T_ROUNDS - 1 - t][:, 64:128])
        for t in range(_T_ROUNDS)
    ]
    return jnp.concatenate(merged, axis=1)


def _phase_a_kernel(prec, R, x_ref, y_ref, w_ref,
                    xp_ref, yp_ref, n1_ref, n2_ref, t1_ref, rs_ref):
    i = pl.program_id(0)

    @pl.when(i == 0)
    def _():
        yp = jnp.dot(y_ref[...], w_ref[...], precision=prec,
                     preferred_element_type=jnp.float32)
        n2 = jnp.sqrt(jnp.sum(yp * yp, axis=1, keepdims=True))
        n2_ref[...] = n2
        yp_ref[...] = yp * (1.0 / jnp.maximum(n2, 1e-30))

    xp_r = jnp.dot(x_ref[...], w_ref[...], precision=prec,
                   preferred_element_type=jnp.float32)
    n1_r = jnp.sqrt(jnp.sum(xp_r * xp_r, axis=1, keepdims=True))
    n1_ref[pl.ds(i * R, R), :] = n1_r
    xn_r = xp_r * (1.0 / jnp.maximum(n1_r, 1e-30))
    xp_ref[...] = xn_r
    cos = _dot_nt(xn_r, yp_ref[...], prec)
    cand = _chunk_candidates(cos, R, cos.shape[1])
    t, rs = _topk_stats(cand, R)
    t1_ref[pl.ds(i * R, R), :] = t
    rs_ref[pl.ds(i * R, R), :] = rs


def _phase_b_kernel(prec, R, c2, xp_ref, yp_ref, stats_ref, dr_ref,
                    s1_ref, s2_ref):
    i = pl.program_id(0)
    stats_r = stats_ref[pl.ds(i * R, R), :]

    xn_r = xp_ref[pl.ds(i * R, R), :]
    cos = _dot_nt(xn_r, yp_ref[...], prec)
    t1_r = stats_r[:, 0:1]
    dc_r = stats_r[:, 1:2]
    keep = cos >= t1_r
    s1_ref[...] = jnp.where(keep, jnp.maximum(cos, 0.0), 0.0) * (dc_r * dr_ref[...])

    yn_r = yp_ref[pl.ds(i * R, R), :]
    cos_t = _dot_nt(yn_r, xp_ref[...], prec)
    cand = _chunk_candidates(cos_t, R, cos_t.shape[1])
    t2, _ = _topk_stats(cand, R)
    s2_ref[...] = jnp.where(cos_t >= t2, 0.0, c2)


def kernel(X, Y, k, W):
    del k
    n, d_in = X.shape
    RA = 256
    R = 256
    prec = jax.lax.Precision.DEFAULT
    f32 = jnp.float32

    full = lambda shape: pl.BlockSpec(shape, lambda i: (0, 0))

    xp, yp, n1, n2, t1, rs = pl.pallas_call(
        lambda *refs: _phase_a_kernel(prec, RA, *refs),
        grid=(n // RA,),
        in_specs=[pl.BlockSpec((RA, d_in), lambda i: (i, 0)),
                  full(Y.shape), full(W.shape)],
        out_specs=[pl.BlockSpec((RA, W.shape[1]), lambda i: (i, 0)),
                   full((n, W.shape[1])),
                   full((n, 1)), full((n, 1)), full((n, 1)), full((n, 1))],
        out_shape=[jax.ShapeDtypeStruct((n, W.shape[1]), f32),
                   jax.ShapeDtypeStruct((n, W.shape[1]), f32),
                   jax.ShapeDtypeStruct((n, 1), f32),
                   jax.ShapeDtypeStruct((n, 1), f32),
                   jax.ShapeDtypeStruct((n, 1), f32),
                   jax.ShapeDtypeStruct((n, 1), f32)],
    )(X, Y, W)

    dcol = rs ** -0.5
    dcol = jnp.where(jnp.isinf(dcol), 0.0, dcol)
    drow = jnp.reshape(dcol, (1, n))
    c2 = float(np.float32(_NEG) / (np.float32(_NEG) * np.float32(n - _K)))

    stats = jnp.concatenate([t1, dcol], axis=1)
    s1, s2 = pl.pallas_call(
        lambda *refs: _phase_b_kernel(prec, R, c2, *refs),
        grid=(n // R,),
        in_specs=[full(xp.shape), full(yp.shape), full((n, 2)), full((1, n))],
        out_specs=[pl.BlockSpec((R, n), lambda i: (i, 0)),
                   pl.BlockSpec((R, n), lambda i: (i, 0))],
        out_shape=[jax.ShapeDtypeStruct((n, n), f32),
                   jax.ShapeDtypeStruct((n, n), f32)],
    )(xp, yp, stats, drow)
    return (s1, s2)
